# baseline (speedup 1.0000x reference)
import jax
import jax.numpy as jnp
from jax import lax
from jax.experimental import pallas as pl

IMG_H = 224
IMG_W = 224
N_PTS = 128
MIN_DEPTH = 0.1
MAX_DEPTH = 8.0
B = 4

N_PER = IMG_H * IMG_W
N_CHUNKS = N_PER // 128
GRID = 14
DIR_ROWS = (N_CHUNKS * B) // GRID
XY_ROWS = (N_CHUNKS * 2) // 7
LEN_ROWS = N_PER // 7


def _f32(x):
    return x.astype(jnp.float32)


def _bf(x):
    return x.astype(jnp.bfloat16).astype(jnp.float32)


def _ndc(idx, last):
    t = _f32(idx) / jnp.float32(last)
    return jnp.where(idx == last, jnp.float32(1.0), t - (jnp.float32(1.0) - t))


def _pix_to_hw(npix):
    m = npix >> 5
    h = (m * 9363) >> 16
    w = npix - h * 224
    return h, w


def _rays_kernel(R_ref, T_ref, f_ref, pp_ref, orig_ref, dir_ref, len_ref, xy_ref):
    s = pl.program_id(0)
    lane = lax.broadcasted_iota(jnp.int32, (1, 128), 1)

    def sel_b(bb, vals):
        return jnp.where(bb == 0, vals[0],
                         jnp.where(bb == 1, vals[1],
                                   jnp.where(bb == 2, vals[2], vals[3])))

    r = lax.broadcasted_iota(jnp.int32, (DIR_ROWS, 1), 0) + DIR_ROWS * s
    k = r >> 2
    bb = r & 3
    npix = (k << 7) + lane
    h, w = _pix_to_hw(npix)

    fx = sel_b(bb, [f_ref[0, i] for i in range(B)])
    fy = sel_b(bb, [f_ref[1, i] for i in range(B)])
    px = sel_b(bb, [pp_ref[0, i] for i in range(B)])
    py = sel_b(bb, [pp_ref[1, i] for i in range(B)])

    dx = (_ndc(w, 223) - px) / fx
    dy = (_ndc(h, 223) - py) / fy
    dxb = _bf(dx)
    dyb = _bf(dy)

    rows = [[sel_b(bb, [R_ref[ci, i, j] for i in range(B)])
             for j in range(3)] for ci in range(3)]

    comp = lax.broadcasted_iota(jnp.int32, (3, 1, 1), 0)

    def sel_c(vals):
        return jnp.where(comp == 0, vals[0][None],
                         jnp.where(comp == 1, vals[1][None], vals[2][None]))

    ra = sel_c([rows[0][0], rows[1][0], rows[2][0]])
    rb = sel_c([rows[0][1], rows[1][1], rows[2][1]])
    rc = sel_c([rows[0][2], rows[1][2], rows[2][2]])
    num = dxb[None] * _bf(ra) + (dyb[None] * _bf(rb) + _bf(rc))
    den = jnp.abs(dxb * _bf(rows[0][0]) + (dyb * _bf(rows[0][1])
                                           + _bf(rows[0][2])))
    dir_ref[...] = num / den[None]

    t = [[T_ref[i, j] for j in range(3)] for i in range(B)]
    rr = [[[R_ref[ci, i, j] for j in range(3)] for ci in range(3)]
          for i in range(B)]
    ctr = [[-(rr[i][ci][0] * t[i][0] + rr[i][ci][1] * t[i][1]
              + rr[i][ci][2] * t[i][2]) for i in range(B)] for ci in range(3)]
    ctr_col = sel_c([sel_b(bb, ctr[0]), sel_b(bb, ctr[1]), sel_b(bb, ctr[2])])
    orig_ref[...] = jnp.broadcast_to(ctr_col, (3, DIR_ROWS, 128))

    r2 = lax.broadcasted_iota(jnp.int32, (XY_ROWS, 1), 0) + XY_ROWS * (s % 7)
    c2 = r2 & 1
    npix2 = ((r2 >> 1) << 7) + lane
    h2, w2 = _pix_to_hw(npix2)
    xy_ref[...] = jnp.broadcast_to(
        jnp.where(c2 == 0, _ndc(w2, 223), _ndc(h2, 223))[None], (2, XY_ROWS, 128))

    dlane = lax.broadcasted_iota(jnp.int32, (1, N_PTS), 1)
    td = _f32(dlane) / jnp.float32(N_PTS - 1)
    depths = jnp.where(dlane == N_PTS - 1, jnp.float32(MAX_DEPTH),
                       jnp.float32(MIN_DEPTH) * (jnp.float32(1.0) - td)
                       + jnp.float32(MAX_DEPTH) * td)
    len_ref[...] = jnp.broadcast_to(depths, (2, LEN_ROWS, N_PTS))


@jax.jit
def kernel(R, T, focal_length, principal_point):
    out_shapes = (
        jax.ShapeDtypeStruct((3, B * N_CHUNKS, 128), jnp.float32),
        jax.ShapeDtypeStruct((3, B * N_CHUNKS, 128), jnp.float32),
        jax.ShapeDtypeStruct((B, N_PER, N_PTS), jnp.float32),
        jax.ShapeDtypeStruct((B, 2 * N_CHUNKS, 128), jnp.float32),
    )
    in_specs = [
        pl.BlockSpec((3, B, 3), lambda s: (0, 0, 0)),
        pl.BlockSpec((B, 3), lambda s: (0, 0)),
        pl.BlockSpec((2, B), lambda s: (0, 0)),
        pl.BlockSpec((2, B), lambda s: (0, 0)),
    ]
    out_specs = (
        pl.BlockSpec((3, DIR_ROWS, 128), lambda s: (0, s, 0)),
        pl.BlockSpec((3, DIR_ROWS, 128), lambda s: (0, s, 0)),
        pl.BlockSpec((2, LEN_ROWS, N_PTS), lambda s: (s // 7, s % 7, 0)),
        pl.BlockSpec((2, XY_ROWS, 128), lambda s: (s // 7, s % 7, 0)),
    )
    origins_p, directions_p, lengths, xys_p = pl.pallas_call(
        _rays_kernel,
        grid=(GRID,),
        in_specs=in_specs,
        out_specs=out_specs,
        out_shape=out_shapes,
    )(R.transpose(1, 0, 2), T, focal_length.T, principal_point.T)

    origins = origins_p.reshape(3, N_CHUNKS, B, 128).transpose(2, 1, 3, 0)
    origins = origins.reshape(B, N_PER, 3)
    directions = directions_p.reshape(3, N_CHUNKS, B, 128).transpose(2, 1, 3, 0)
    directions = directions.reshape(B, N_PER, 3)
    xys = xys_p.reshape(B, N_CHUNKS, 2, 128).transpose(0, 1, 3, 2)
    xys = xys.reshape(B, N_PER, 2)
    return (origins, directions, lengths, xys)


# speedup vs baseline: 1.0256x; 1.0256x over previous
import jax
import jax.numpy as jnp
from jax import lax
from jax.experimental import pallas as pl

IMG_H = 224
IMG_W = 224
N_PTS = 128
MIN_DEPTH = 0.1
MAX_DEPTH = 8.0
B = 4

N_PER = IMG_H * IMG_W
N_CHUNKS = N_PER // 128
GRID = 28
DIR_ROWS = (N_CHUNKS * B) // GRID
XY_ROWS = (N_CHUNKS * 2) // (GRID // B)
LEN_ROWS = N_PER // (GRID // B)


def _f32(x):
    return x.astype(jnp.float32)


def _bf(x):
    return x.astype(jnp.bfloat16).astype(jnp.float32)


def _ndc(idx, last):
    t = _f32(idx) / jnp.float32(last)
    return jnp.where(idx == last, jnp.float32(1.0), t - (jnp.float32(1.0) - t))


def _pix_to_hw(npix):
    m = npix >> 5
    h = (m * 9363) >> 16
    w = npix - h * 224
    return h, w


def _rays_kernel(R_ref, T_ref, f_ref, pp_ref, orig_ref, dir_ref, len_ref, xy_ref):
    s = pl.program_id(0)
    lane = lax.broadcasted_iota(jnp.int32, (1, 128), 1)

    def sel_b(bb, vals):
        return jnp.where(bb == 0, vals[0],
                         jnp.where(bb == 1, vals[1],
                                   jnp.where(bb == 2, vals[2], vals[3])))

    r = lax.broadcasted_iota(jnp.int32, (DIR_ROWS, 1), 0) + DIR_ROWS * s
    k = r >> 2
    bb = r & 3
    npix = (k << 7) + lane
    h, w = _pix_to_hw(npix)

    fx = sel_b(bb, [f_ref[0, i] for i in range(B)])
    fy = sel_b(bb, [f_ref[1, i] for i in range(B)])
    px = sel_b(bb, [pp_ref[0, i] for i in range(B)])
    py = sel_b(bb, [pp_ref[1, i] for i in range(B)])

    dx = (_ndc(w, 223) - px) / fx
    dy = (_ndc(h, 223) - py) / fy
    dxb = _bf(dx)
    dyb = _bf(dy)

    rows = [[sel_b(bb, [R_ref[ci, i, j] for i in range(B)])
             for j in range(3)] for ci in range(3)]

    comp = lax.broadcasted_iota(jnp.int32, (3, 1, 1), 0)

    def sel_c(vals):
        return jnp.where(comp == 0, vals[0][None],
                         jnp.where(comp == 1, vals[1][None], vals[2][None]))

    ra = sel_c([rows[0][0], rows[1][0], rows[2][0]])
    rb = sel_c([rows[0][1], rows[1][1], rows[2][1]])
    rc = sel_c([rows[0][2], rows[1][2], rows[2][2]])
    num = dxb[None] * _bf(ra) + (dyb[None] * _bf(rb) + _bf(rc))
    den = jnp.abs(dxb * _bf(rows[0][0]) + (dyb * _bf(rows[0][1])
                                           + _bf(rows[0][2])))
    dir_ref[...] = num / den[None]

    t = [[T_ref[i, j] for j in range(3)] for i in range(B)]
    rr = [[[R_ref[ci, i, j] for j in range(3)] for ci in range(3)]
          for i in range(B)]
    ctr = [[-(rr[i][ci][0] * t[i][0] + rr[i][ci][1] * t[i][1]
              + rr[i][ci][2] * t[i][2]) for i in range(B)] for ci in range(3)]
    ctr_col = sel_c([sel_b(bb, ctr[0]), sel_b(bb, ctr[1]), sel_b(bb, ctr[2])])
    orig_ref[...] = jnp.broadcast_to(ctr_col, (3, DIR_ROWS, 128))

    r2 = lax.broadcasted_iota(jnp.int32, (XY_ROWS, 1), 0) + XY_ROWS * (s % 7)
    c2 = r2 & 1
    npix2 = ((r2 >> 1) << 7) + lane
    h2, w2 = _pix_to_hw(npix2)
    xy_ref[0] = jnp.where(c2 == 0, _ndc(w2, 223), _ndc(h2, 223))

    dlane = lax.broadcasted_iota(jnp.int32, (1, N_PTS), 1)
    td = _f32(dlane) / jnp.float32(N_PTS - 1)
    depths = jnp.where(dlane == N_PTS - 1, jnp.float32(MAX_DEPTH),
                       jnp.float32(MIN_DEPTH) * (jnp.float32(1.0) - td)
                       + jnp.float32(MAX_DEPTH) * td)
    len_ref[0] = jnp.broadcast_to(depths, (LEN_ROWS, N_PTS))


@jax.jit
def kernel(R, T, focal_length, principal_point):
    out_shapes = (
        jax.ShapeDtypeStruct((3, B * N_CHUNKS, 128), jnp.float32),
        jax.ShapeDtypeStruct((3, B * N_CHUNKS, 128), jnp.float32),
        jax.ShapeDtypeStruct((B, N_PER, N_PTS), jnp.float32),
        jax.ShapeDtypeStruct((B, 2 * N_CHUNKS, 128), jnp.float32),
    )
    in_specs = [
        pl.BlockSpec((3, B, 3), lambda s: (0, 0, 0)),
        pl.BlockSpec((B, 3), lambda s: (0, 0)),
        pl.BlockSpec((2, B), lambda s: (0, 0)),
        pl.BlockSpec((2, B), lambda s: (0, 0)),
    ]
    out_specs = (
        pl.BlockSpec((3, DIR_ROWS, 128), lambda s: (0, s, 0)),
        pl.BlockSpec((3, DIR_ROWS, 128), lambda s: (0, s, 0)),
        pl.BlockSpec((1, LEN_ROWS, N_PTS), lambda s: (s // 7, s % 7, 0)),
        pl.BlockSpec((1, XY_ROWS, 128), lambda s: (s // 7, s % 7, 0)),
    )
    origins_p, directions_p, lengths, xys_p = pl.pallas_call(
        _rays_kernel,
        grid=(GRID,),
        in_specs=in_specs,
        out_specs=out_specs,
        out_shape=out_shapes,
    )(R.transpose(1, 0, 2), T, focal_length.T, principal_point.T)

    origins = origins_p.reshape(3, N_CHUNKS, B, 128).transpose(2, 1, 3, 0)
    origins = origins.reshape(B, N_PER, 3)
    directions = directions_p.reshape(3, N_CHUNKS, B, 128).transpose(2, 1, 3, 0)
    directions = directions.reshape(B, N_PER, 3)
    xys = xys_p.reshape(B, N_CHUNKS, 2, 128).transpose(0, 1, 3, 2)
    xys = xys.reshape(B, N_PER, 2)
    return (origins, directions, lengths, xys)
